# baseline (speedup 1.0000x reference)
import jax
import jax.numpy as jnp
from jax import lax
from jax.experimental import pallas as pl
from jax.experimental.pallas import tpu as pltpu
from jax.experimental.pallas import tpu_sc as plsc

N = 10000
E = 320000
F = 128
D3 = 3 * F

NC = 2
NS = 16
NW = NC * NS
EPW = E // NW
CHUNK = 40
NCHUNK = EPW // CHUNK
TCHUNK = E // CHUNK
ROWS_PER_TILE = 624
TAIL_ROWS = N - NS * ROWS_PER_TILE
TAIL_R0 = NS * ROWS_PER_TILE


def _mm_body(x_ref, w_ref, o_ref):
    o_ref[...] = jnp.dot(x_ref[...], w_ref[...],
                         preferred_element_type=jnp.float32)


def _support_matmul(x2d, wcat):
    BN = 1000
    return pl.pallas_call(
        _mm_body,
        grid=(N // BN,),
        in_specs=[pl.BlockSpec((BN, F), lambda i: (i, 0)),
                  pl.BlockSpec((F, D3), lambda i: (0, 0))],
        out_specs=pl.BlockSpec((BN, D3), lambda i: (i, 0)),
        out_shape=jax.ShapeDtypeStruct((N, D3), jnp.float32),
    )(x2d, wcat)


def _sc_body(eidx_hbm, ews_hbm, s_hbm, zero_hbm,
             out_hbm, eidx_v, ews_v, rows_v, msg_v, acc_sh, semi, semg):
    cid = lax.axis_index("c")
    sid = lax.axis_index("s")
    wid = sid * NC + cid

    r0 = sid * ROWS_PER_TILE
    pltpu.sync_copy(zero_hbm.at[pl.ds(r0, ROWS_PER_TILE)],
                    acc_sh.at[pl.ds(r0, ROWS_PER_TILE)])

    @pl.when(sid == 0)
    def _():
        pltpu.sync_copy(zero_hbm.at[pl.ds(TAIL_R0, TAIL_ROWS)],
                        acc_sh.at[pl.ds(TAIL_R0, TAIL_ROWS)])

    plsc.subcore_barrier()

    crow0 = wid * NCHUNK

    def issue_idx(j, b):
        pltpu.async_copy(eidx_hbm.at[crow0 + j], eidx_v.at[b], semi.at[b])
        pltpu.async_copy(ews_hbm.at[crow0 + j], ews_v.at[b], semi.at[b])

    def wait_idx(j, b):
        pltpu.make_async_copy(eidx_hbm.at[crow0 + j], eidx_v.at[b],
                              semi.at[b]).wait()
        pltpu.make_async_copy(ews_hbm.at[crow0 + j], ews_v.at[b],
                              semi.at[b]).wait()

    def issue_gather(b):
        pltpu.async_copy(s_hbm.at[eidx_v.at[b, 0]], rows_v.at[b], semg.at[b])

    def wait_gather(b):
        pltpu.make_async_copy(s_hbm.at[eidx_v.at[b, 0]], rows_v.at[b],
                              semg.at[b]).wait()

    def compute(b):
        z16 = jnp.zeros((16,), jnp.int32)

        @plsc.parallel_loop(0, CHUNK, 1)
        def _(e):
            ws = ews_v[b, e, pl.ds(0, 16)]
            w1 = ws.at[jnp.zeros((16,), jnp.int32)].get(
                mode="promise_in_bounds")
            w2 = ws.at[jnp.full((16,), 1, jnp.int32)].get(
                mode="promise_in_bounds")
            w3 = ws.at[jnp.full((16,), 2, jnp.int32)].get(
                mode="promise_in_bounds")
            for f in range(F // 16):
                a = rows_v[b, e, pl.ds(f * 16, 16)]
                bb = rows_v[b, e, pl.ds(F + f * 16, 16)]
                cc = rows_v[b, e, pl.ds(2 * F + f * 16, 16)]
                msg_v[e, pl.ds(f * 16, 16)] = w1 * a + w2 * bb + w3 * cc

    issue_idx(0, 0)
    issue_idx(1, 1)
    wait_idx(0, 0)
    issue_gather(0)

    def pair_body(p, carry):
        j0 = p * 2
        for b in range(2):
            j = j0 + b
            nb = 1 - b

            @pl.when(j + 1 < NCHUNK)
            def _():
                wait_idx(j + 1, nb)
                issue_gather(nb)

            wait_gather(b)
            compute(b)
            pltpu.sync_copy(msg_v, acc_sh.at[eidx_v.at[b, 1]], add=True)

            @pl.when(j + 2 < NCHUNK)
            def _():
                issue_idx(j + 2, b)
        return carry

    lax.fori_loop(0, NCHUNK // 2, pair_body, 0)

    plsc.subcore_barrier()
    pltpu.sync_copy(acc_sh.at[pl.ds(r0, ROWS_PER_TILE)],
                    out_hbm.at[cid, pl.ds(r0, ROWS_PER_TILE)])

    @pl.when(sid == 0)
    def _():
        pltpu.sync_copy(acc_sh.at[pl.ds(TAIL_R0, TAIL_ROWS)],
                        out_hbm.at[cid, pl.ds(TAIL_R0, TAIL_ROWS)])


def _sc_spmm(eidx, ews, s, zeros):
    mesh = plsc.VectorSubcoreMesh(core_axis_name="c", subcore_axis_name="s")
    return pl.kernel(
        _sc_body,
        out_type=jax.ShapeDtypeStruct((NC, N, F), jnp.float32),
        mesh=mesh,
        scratch_types=[
            pltpu.VMEM((2, 2, CHUNK), jnp.int32),
            pltpu.VMEM((2, CHUNK, 16), jnp.float32),
            pltpu.VMEM((2, CHUNK, D3), jnp.float32),
            pltpu.VMEM((CHUNK, F), jnp.float32),
            pltpu.VMEM_SHARED((N, F), jnp.float32),
            pltpu.SemaphoreType.DMA((2,)),
            pltpu.SemaphoreType.DMA((2,)),
        ],
    )(eidx, ews, s, zeros)


def _epi_body(p_ref, b_ref, o_ref):
    o_ref[...] = jnp.maximum(p_ref[0] + p_ref[1] + b_ref[...], 0.0)


def _epilogue(partials, bias2d):
    BN = 1000
    return pl.pallas_call(
        _epi_body,
        grid=(N // BN,),
        in_specs=[pl.BlockSpec((NC, BN, F), lambda i: (0, i, 0)),
                  pl.BlockSpec((1, F), lambda i: (0, 0))],
        out_specs=pl.BlockSpec((BN, F), lambda i: (i, 0)),
        out_shape=jax.ShapeDtypeStruct((N, F), jnp.float32),
    )(partials, bias2d)


def kernel(x, edge_index, ew_1, ew_2, ew_3, W_1, W_2, W_3, bias):
    x2d = x[0]
    wcat = jnp.concatenate([W_1, W_2, W_3], axis=1)
    s = _support_matmul(x2d, wcat)
    src = edge_index[1].reshape(TCHUNK, 1, CHUNK)
    dst = edge_index[0].reshape(TCHUNK, 1, CHUNK)
    eidx = jnp.concatenate([src, dst], axis=1)
    ews3 = jnp.stack([ew_1.reshape(TCHUNK, CHUNK),
                      ew_2.reshape(TCHUNK, CHUNK),
                      ew_3.reshape(TCHUNK, CHUNK)], axis=2)
    ews = jnp.concatenate(
        [ews3, jnp.zeros((TCHUNK, CHUNK, 13), jnp.float32)], axis=2)
    zeros = jnp.zeros((N, F), dtype=jnp.float32)
    partials = _sc_spmm(eidx, ews, s, zeros)
    out = _epilogue(partials, bias.reshape(1, F))
    return out[None]


# speedup vs baseline: 1.2825x; 1.2825x over previous
import jax
import jax.numpy as jnp
from jax import lax
from jax.experimental import pallas as pl
from jax.experimental.pallas import tpu as pltpu
from jax.experimental.pallas import tpu_sc as plsc

N = 10000
E = 320000
F = 128
D3 = 3 * F

NC = 2
NS = 16
NW = NC * NS
EPW = E // NW
CHUNK = 40
NCHUNK = EPW // CHUNK
TCHUNK = E // CHUNK
ROWS_PER_TILE = 624
TAIL_ROWS = N - NS * ROWS_PER_TILE
TAIL_R0 = NS * ROWS_PER_TILE


def _mm_body(x_ref, w_ref, o_ref):
    o_ref[...] = jnp.dot(x_ref[...], w_ref[...],
                         preferred_element_type=jnp.float32)


def _support_matmul(x2d, wcat):
    BN = 1000
    return pl.pallas_call(
        _mm_body,
        grid=(N // BN,),
        in_specs=[pl.BlockSpec((BN, F), lambda i: (i, 0)),
                  pl.BlockSpec((F, D3), lambda i: (0, 0))],
        out_specs=pl.BlockSpec((BN, D3), lambda i: (i, 0)),
        out_shape=jax.ShapeDtypeStruct((N, D3), jnp.float32),
    )(x2d, wcat)


def _sc_body(eidx_hbm, ews_hbm, s_hbm, zero_hbm,
             out_hbm, eidx_v, dstc_v, ews_v, rows_v, msg_v, acc_sh,
             semi, semg, sems):
    cid = lax.axis_index("c")
    sid = lax.axis_index("s")
    wid = sid * NC + cid

    r0 = sid * ROWS_PER_TILE
    pltpu.sync_copy(zero_hbm.at[pl.ds(r0, ROWS_PER_TILE)],
                    acc_sh.at[pl.ds(r0, ROWS_PER_TILE)])

    @pl.when(sid == 0)
    def _():
        pltpu.sync_copy(zero_hbm.at[pl.ds(TAIL_R0, TAIL_ROWS)],
                        acc_sh.at[pl.ds(TAIL_R0, TAIL_ROWS)])

    plsc.subcore_barrier()

    crow0 = wid * NCHUNK

    def issue_idx(j, b):
        pltpu.async_copy(eidx_hbm.at[crow0 + j], eidx_v.at[b], semi.at[b])
        pltpu.async_copy(ews_hbm.at[crow0 + j], ews_v.at[b], semi.at[b])

    def wait_idx(j, b):
        pltpu.make_async_copy(eidx_hbm.at[crow0 + j], eidx_v.at[b],
                              semi.at[b]).wait()
        pltpu.make_async_copy(ews_hbm.at[crow0 + j], ews_v.at[b],
                              semi.at[b]).wait()

    def issue_gather(b):
        pltpu.async_copy(s_hbm.at[eidx_v.at[b, 0]], rows_v.at[b], semg.at[b])

    def wait_gather(b):
        pltpu.make_async_copy(s_hbm.at[eidx_v.at[b, 0]], rows_v.at[b],
                              semg.at[b]).wait()

    def issue_scatter(b):
        pltpu.async_copy(msg_v.at[b], acc_sh.at[dstc_v.at[b, 0]],
                         sems.at[b], add=True)

    def wait_scatter(b):
        pltpu.make_async_copy(msg_v.at[b], acc_sh.at[dstc_v.at[b, 0]],
                              sems.at[b]).wait()

    def compute(b):
        def do_lanes(off, l0, w1v, w2v, w3v):
            for lane in range(l0, 16):
                e = off + lane
                w1 = jnp.full((16,), w1v[lane], jnp.float32)
                w2 = jnp.full((16,), w2v[lane], jnp.float32)
                w3 = jnp.full((16,), w3v[lane], jnp.float32)
                for f in range(F // 16):
                    a = rows_v[b, e, pl.ds(f * 16, 16)]
                    bb = rows_v[b, e, pl.ds(F + f * 16, 16)]
                    cc = rows_v[b, e, pl.ds(2 * F + f * 16, 16)]
                    msg_v[b, e, pl.ds(f * 16, 16)] = (
                        w1 * a + w2 * bb + w3 * cc)

        def group_body(g, carry):
            e0 = g * 16
            do_lanes(e0, 0,
                     ews_v[b, 0, pl.ds(e0, 16)],
                     ews_v[b, 1, pl.ds(e0, 16)],
                     ews_v[b, 2, pl.ds(e0, 16)])
            return carry

        lax.fori_loop(0, 2, group_body, 0)
        do_lanes(24, 8,
                 ews_v[b, 0, pl.ds(24, 16)],
                 ews_v[b, 1, pl.ds(24, 16)],
                 ews_v[b, 2, pl.ds(24, 16)])
        for off in (0, 16, 24):
            dstc_v[b, 0, pl.ds(off, 16)] = eidx_v[b, 1, pl.ds(off, 16)]

    issue_idx(0, 0)
    issue_idx(1, 1)
    wait_idx(0, 0)
    issue_gather(0)

    def pair_body(p, carry):
        j0 = p * 2
        for b in range(2):
            j = j0 + b
            nb = 1 - b

            @pl.when(j + 1 < NCHUNK)
            def _():
                wait_idx(j + 1, nb)
                issue_gather(nb)

            wait_gather(b)

            @pl.when(j >= 2)
            def _():
                wait_scatter(b)

            compute(b)
            issue_scatter(b)

            @pl.when(j + 2 < NCHUNK)
            def _():
                issue_idx(j + 2, b)
        return carry

    lax.fori_loop(0, NCHUNK // 2, pair_body, 0)
    wait_scatter(0)
    wait_scatter(1)

    plsc.subcore_barrier()
    pltpu.sync_copy(acc_sh.at[pl.ds(r0, ROWS_PER_TILE)],
                    out_hbm.at[cid, pl.ds(r0, ROWS_PER_TILE)])

    @pl.when(sid == 0)
    def _():
        pltpu.sync_copy(acc_sh.at[pl.ds(TAIL_R0, TAIL_ROWS)],
                        out_hbm.at[cid, pl.ds(TAIL_R0, TAIL_ROWS)])


def _sc_spmm(eidx, ews, s, zeros):
    mesh = plsc.VectorSubcoreMesh(core_axis_name="c", subcore_axis_name="s")
    return pl.kernel(
        _sc_body,
        out_type=jax.ShapeDtypeStruct((NC, N, F), jnp.float32),
        mesh=mesh,
        scratch_types=[
            pltpu.VMEM((2, 2, CHUNK), jnp.int32),
            pltpu.VMEM((2, 1, CHUNK), jnp.int32),
            pltpu.VMEM((2, 3, CHUNK), jnp.float32),
            pltpu.VMEM((2, CHUNK, D3), jnp.float32),
            pltpu.VMEM((2, CHUNK, F), jnp.float32),
            pltpu.VMEM_SHARED((N, F), jnp.float32),
            pltpu.SemaphoreType.DMA((2,)),
            pltpu.SemaphoreType.DMA((2,)),
            pltpu.SemaphoreType.DMA((2,)),
        ],
    )(eidx, ews, s, zeros)


def _epi_body(p_ref, b_ref, o_ref):
    o_ref[...] = jnp.maximum(p_ref[0] + p_ref[1] + b_ref[...], 0.0)


def _epilogue(partials, bias2d):
    BN = 1000
    return pl.pallas_call(
        _epi_body,
        grid=(N // BN,),
        in_specs=[pl.BlockSpec((NC, BN, F), lambda i: (0, i, 0)),
                  pl.BlockSpec((1, F), lambda i: (0, 0))],
        out_specs=pl.BlockSpec((BN, F), lambda i: (i, 0)),
        out_shape=jax.ShapeDtypeStruct((N, F), jnp.float32),
    )(partials, bias2d)


def kernel(x, edge_index, ew_1, ew_2, ew_3, W_1, W_2, W_3, bias):
    x2d = x[0]
    wcat = jnp.concatenate([W_1, W_2, W_3], axis=1)
    s = _support_matmul(x2d, wcat)
    src = edge_index[1].reshape(TCHUNK, 1, CHUNK)
    dst = edge_index[0].reshape(TCHUNK, 1, CHUNK)
    eidx = jnp.concatenate([src, dst], axis=1)
    ews = jnp.stack([ew_1.reshape(TCHUNK, CHUNK),
                     ew_2.reshape(TCHUNK, CHUNK),
                     ew_3.reshape(TCHUNK, CHUNK)], axis=1)
    zeros = jnp.zeros((N, F), dtype=jnp.float32)
    partials = _sc_spmm(eidx, ews, s, zeros)
    out = _epilogue(partials, bias.reshape(1, F))
    return out[None]


# speedup vs baseline: 1.4704x; 1.1465x over previous
import jax
import jax.numpy as jnp
from jax import lax
from jax.experimental import pallas as pl
from jax.experimental.pallas import tpu as pltpu
from jax.experimental.pallas import tpu_sc as plsc

N = 10000
E = 320000
F = 128
D3 = 3 * F

NC = 2
NS = 16
NW = NC * NS
EPW = E // NW
CHUNK = 40
NCHUNK = EPW // CHUNK
TCHUNK = E // CHUNK
ROWS_PER_TILE = 624
TAIL_ROWS = N - NS * ROWS_PER_TILE
TAIL_R0 = NS * ROWS_PER_TILE


def _mm_body(x_ref, w_ref, o_ref):
    o_ref[...] = jnp.dot(x_ref[...], w_ref[...],
                         preferred_element_type=jnp.float32)


def _support_matmul(x2d, wcat):
    BN = 1000
    return pl.pallas_call(
        _mm_body,
        grid=(N // BN,),
        in_specs=[pl.BlockSpec((BN, F), lambda i: (i, 0)),
                  pl.BlockSpec((F, D3), lambda i: (0, 0))],
        out_specs=pl.BlockSpec((BN, D3), lambda i: (i, 0)),
        out_shape=jax.ShapeDtypeStruct((N, D3), jnp.float32),
    )(x2d, wcat)


def _sc_body(eidx_hbm, ews_hbm, s_hbm, zero_hbm,
             out_hbm, eidx_v, dstc_v, ews_v, rows_v, msg_v, acc_sh,
             semie, semw, semg, sems):
    cid = lax.axis_index("c")
    sid = lax.axis_index("s")
    wid = sid * NC + cid

    r0 = sid * ROWS_PER_TILE
    pltpu.sync_copy(zero_hbm.at[pl.ds(r0, ROWS_PER_TILE)],
                    acc_sh.at[pl.ds(r0, ROWS_PER_TILE)])

    @pl.when(sid == 0)
    def _():
        pltpu.sync_copy(zero_hbm.at[pl.ds(TAIL_R0, TAIL_ROWS)],
                        acc_sh.at[pl.ds(TAIL_R0, TAIL_ROWS)])

    plsc.subcore_barrier()

    crow0 = wid * NCHUNK

    def issue_eidx(j, b):
        pltpu.async_copy(eidx_hbm.at[crow0 + j], eidx_v.at[b], semie.at[b])

    def wait_eidx(j, b):
        pltpu.make_async_copy(eidx_hbm.at[crow0 + j], eidx_v.at[b],
                              semie.at[b]).wait()

    def issue_ews(j, b):
        pltpu.async_copy(ews_hbm.at[crow0 + j], ews_v.at[b], semw.at[b])

    def wait_ews(j, b):
        pltpu.make_async_copy(ews_hbm.at[crow0 + j], ews_v.at[b],
                              semw.at[b]).wait()

    def issue_gather(b):
        pltpu.async_copy(s_hbm.at[eidx_v.at[b, 0]], rows_v.at[b], semg.at[b])

    def wait_gather(b):
        pltpu.make_async_copy(s_hbm.at[eidx_v.at[b, 0]], rows_v.at[b],
                              semg.at[b]).wait()

    def issue_scatter(b):
        pltpu.async_copy(msg_v.at[b], acc_sh.at[dstc_v.at[b, 0]],
                         sems.at[b], add=True)

    def wait_scatter(b):
        pltpu.make_async_copy(msg_v.at[b], acc_sh.at[dstc_v.at[b, 0]],
                              sems.at[b]).wait()

    def compute(b):
        def do_lanes(off, l0, w1v, w2v, w3v):
            for lane in range(l0, 16):
                e = off + lane
                w1 = jnp.full((16,), w1v[lane], jnp.float32)
                w2 = jnp.full((16,), w2v[lane], jnp.float32)
                w3 = jnp.full((16,), w3v[lane], jnp.float32)
                for f in range(F // 16):
                    a = rows_v[b, e, pl.ds(f * 16, 16)]
                    bb = rows_v[b, e, pl.ds(F + f * 16, 16)]
                    cc = rows_v[b, e, pl.ds(2 * F + f * 16, 16)]
                    msg_v[b, e, pl.ds(f * 16, 16)] = (
                        w1 * a + w2 * bb + w3 * cc)

        def group_body(g, carry):
            e0 = g * 16
            do_lanes(e0, 0,
                     ews_v[b, 0, pl.ds(e0, 16)],
                     ews_v[b, 1, pl.ds(e0, 16)],
                     ews_v[b, 2, pl.ds(e0, 16)])
            return carry

        lax.fori_loop(0, 2, group_body, 0)
        do_lanes(24, 8,
                 ews_v[b, 0, pl.ds(24, 16)],
                 ews_v[b, 1, pl.ds(24, 16)],
                 ews_v[b, 2, pl.ds(24, 16)])
    def copy_dst(b):
        for off in (0, 16, 24):
            dstc_v[b, 0, pl.ds(off, 16)] = eidx_v[b, 1, pl.ds(off, 16)]

    issue_eidx(0, 0)
    issue_eidx(1, 1)
    issue_ews(0, 0)
    issue_ews(1, 1)
    wait_eidx(0, 0)
    issue_gather(0)

    def pair_body(p, carry):
        j0 = p * 2
        for b in range(2):
            j = j0 + b
            nb = 1 - b

            @pl.when(j >= 2)
            def _():
                wait_scatter(b)

            copy_dst(b)
            wait_gather(b)

            @pl.when(j + 2 < NCHUNK)
            def _():
                issue_eidx(j + 2, b)

            @pl.when(j + 1 < NCHUNK)
            def _():
                wait_eidx(j + 1, nb)
                issue_gather(nb)

            wait_ews(j, b)
            compute(b)
            issue_scatter(b)

            @pl.when(j + 2 < NCHUNK)
            def _():
                issue_ews(j + 2, b)
        return carry

    lax.fori_loop(0, NCHUNK // 2, pair_body, 0)
    wait_scatter(0)
    wait_scatter(1)

    plsc.subcore_barrier()
    pltpu.sync_copy(acc_sh.at[pl.ds(r0, ROWS_PER_TILE)],
                    out_hbm.at[cid, pl.ds(r0, ROWS_PER_TILE)])

    @pl.when(sid == 0)
    def _():
        pltpu.sync_copy(acc_sh.at[pl.ds(TAIL_R0, TAIL_ROWS)],
                        out_hbm.at[cid, pl.ds(TAIL_R0, TAIL_ROWS)])


def _sc_spmm(eidx, ews, s, zeros):
    mesh = plsc.VectorSubcoreMesh(core_axis_name="c", subcore_axis_name="s")
    return pl.kernel(
        _sc_body,
        out_type=jax.ShapeDtypeStruct((NC, N, F), jnp.float32),
        mesh=mesh,
        scratch_types=[
            pltpu.VMEM((2, 2, CHUNK), jnp.int32),
            pltpu.VMEM((2, 1, CHUNK), jnp.int32),
            pltpu.VMEM((2, 3, CHUNK), jnp.float32),
            pltpu.VMEM((2, CHUNK, D3), jnp.float32),
            pltpu.VMEM((2, CHUNK, F), jnp.float32),
            pltpu.VMEM_SHARED((N, F), jnp.float32),
            pltpu.SemaphoreType.DMA((2,)),
            pltpu.SemaphoreType.DMA((2,)),
            pltpu.SemaphoreType.DMA((2,)),
            pltpu.SemaphoreType.DMA((2,)),
        ],
    )(eidx, ews, s, zeros)


def _epi_body(p_ref, b_ref, o_ref):
    o_ref[...] = jnp.maximum(p_ref[0] + p_ref[1] + b_ref[...], 0.0)


def _epilogue(partials, bias2d):
    BN = 1000
    return pl.pallas_call(
        _epi_body,
        grid=(N // BN,),
        in_specs=[pl.BlockSpec((NC, BN, F), lambda i: (0, i, 0)),
                  pl.BlockSpec((1, F), lambda i: (0, 0))],
        out_specs=pl.BlockSpec((BN, F), lambda i: (i, 0)),
        out_shape=jax.ShapeDtypeStruct((N, F), jnp.float32),
    )(partials, bias2d)


def kernel(x, edge_index, ew_1, ew_2, ew_3, W_1, W_2, W_3, bias):
    x2d = x[0]
    wcat = jnp.concatenate([W_1, W_2, W_3], axis=1)
    s = _support_matmul(x2d, wcat)
    src = edge_index[1].reshape(TCHUNK, 1, CHUNK)
    dst = edge_index[0].reshape(TCHUNK, 1, CHUNK)
    eidx = jnp.concatenate([src, dst], axis=1)
    ews = jnp.stack([ew_1.reshape(TCHUNK, CHUNK),
                     ew_2.reshape(TCHUNK, CHUNK),
                     ew_3.reshape(TCHUNK, CHUNK)], axis=1)
    zeros = jnp.zeros((N, F), dtype=jnp.float32)
    partials = _sc_spmm(eidx, ews, s, zeros)
    out = _epilogue(partials, bias.reshape(1, F))
    return out[None]
